# Initial kernel scaffold; baseline (speedup 1.0000x reference)
import jax
import jax.numpy as jnp
from jax.experimental import pallas as pl


def kernel(v, emb_table, gamma, beta):
    raise NotImplementedError("write your pallas kernel here")


# speedup vs baseline: 15.7439x; 15.7439x over previous
import jax
import jax.numpy as jnp
from jax.experimental import pallas as pl

EPS = 1e-5
B, L, D = 64, 4096, 128
NTOK = 5
RPB = L // D


def _stats_body(v_ref, emb_ref, gam_ref, bet_ref, tn_ref):
    vf = v_ref[...]
    n = float(B * L)
    emb = emb_ref[...]
    counts = [jnp.sum((vf == t).astype(jnp.float32)) for t in range(NTOK)]
    mean = counts[0] * emb[0:1]
    ex2 = counts[0] * (emb[0:1] * emb[0:1])
    for t in range(1, NTOK):
        mean = mean + counts[t] * emb[t:t + 1]
        ex2 = ex2 + counts[t] * (emb[t:t + 1] * emb[t:t + 1])
    mean = mean / n
    ex2 = ex2 / n
    var = ex2 - mean * mean
    scale = gam_ref[...] * jax.lax.rsqrt(var + EPS)
    shift = bet_ref[...] - mean * scale
    tn_ref[...] = emb * scale + shift


def _dense_body(v_ref, tn_ref, out_ref):
    vm = v_ref[0]
    masks = [(vm == t).astype(jnp.float32) for t in range(NTOK)]
    for c in range(D):
        acc = masks[0] * tn_ref[0, c]
        for t in range(1, NTOK):
            acc = acc + masks[t] * tn_ref[t, c]
        out_ref[0, c * RPB:(c + 1) * RPB, :] = acc


def kernel(v, emb_table, gamma, beta):
    v = v.astype(jnp.int32)
    emb8 = jnp.zeros((8, 128), jnp.float32).at[:NTOK].set(emb_table)
    tn = pl.pallas_call(
        _stats_body,
        out_shape=jax.ShapeDtypeStruct((8, 128), jnp.float32),
    )(v, emb8, gamma.reshape(1, 128), beta.reshape(1, 128))

    v3 = v.reshape(B, RPB, 128)
    out = pl.pallas_call(
        _dense_body,
        grid=(B,),
        in_specs=[
            pl.BlockSpec((1, RPB, 128), lambda b: (b, 0, 0)),
            pl.BlockSpec((8, 128), lambda b: (0, 0)),
        ],
        out_specs=pl.BlockSpec((1, L, 128), lambda b: (b, 0, 0)),
        out_shape=jax.ShapeDtypeStruct((B, L, 128), jnp.float32),
    )(v3, tn)
    return out
